# baseline (speedup 1.0000x reference)
import functools

import jax
import jax.numpy as jnp
from jax.experimental import pallas as pl
from jax.experimental.pallas import tpu as pltpu

_DIM = 3
_THRESHOLD = 0.4
_J = 3
_N = 2048
_CHUNK = 256
_NB = 8


def _build_thunks(xt, w_ref, rdeg_ref):
    acc = {"deg": jnp.zeros((1, _N), jnp.float32)}
    x_rows = [xt[d:d + 1, :] for d in range(_DIM)]

    def make(c):
        def thunk():
            sl = slice(c * _CHUNK, (c + 1) * _CHUNK)
            cols = [x_rows[d][:, sl].T for d in range(_DIM)]
            d0 = cols[0] - x_rows[0]
            D = d0 * d0
            for d in range(1, _DIM):
                dd = cols[d] - x_rows[d]
                D = D + dd * dd
            Wc = jnp.exp(-D)
            Wc = jnp.where(Wc >= _THRESHOLD, Wc, 0.0)
            acc["deg"] = acc["deg"] + jnp.sum(Wc, axis=0, keepdims=True)
            w_ref[sl, :] = Wc.astype(jnp.bfloat16)
        return thunk

    def finalize():
        rdeg_ref[...] = 1.0 / jnp.maximum(acc["deg"], 1e-12)

    return [make(c) for c in range(_N // _CHUNK)] + [finalize]


def _feats(xt, w_ref, rdeg_ref, interleave):
    rdeg_row = rdeg_ref[...]
    steps = list(interleave)

    def apply(s):
        sp = (s * rdeg_row).astype(jnp.bfloat16)
        mv = jax.lax.dot_general(
            sp, w_ref[...], (((1,), (0,)), ((), ())),
            preferred_element_type=jnp.float32)
        out = 0.5 * (s + mv)
        if steps:
            steps.pop(0)()
        return out

    zeros9 = jnp.zeros((3 * _J, _N), jnp.float32)
    s_ = jnp.concatenate([xt, zeros9], axis=0)
    s_ = apply(s_)
    s1 = s_[0:3]
    u0 = jnp.abs(xt - s1)
    s_ = jnp.concatenate([s_[0:3], u0, s_[6:12]], axis=0)
    s_ = apply(s_)
    s2 = s_[0:3]
    u0s1 = s_[3:6]
    u1 = jnp.abs(s1 - s2)
    s_ = jnp.concatenate([s_[0:6], u1, s_[9:12]], axis=0)
    s_ = apply(s_)
    u0s2 = s_[3:6]
    s_ = apply(s_)
    s4, u1s2 = s_[0:3], s_[6:9]
    u2 = jnp.abs(s2 - s4)
    s_ = jnp.concatenate([s_[0:9], u2], axis=0)
    s_ = apply(s_)
    u0s4 = s_[3:6]
    s_ = apply(s_)
    u1s4 = s_[6:9]
    s_ = apply(s_)
    s_ = apply(s_)
    s8, u2s4 = s_[0:3], s_[9:12]
    s_ = apply(s_)
    u0s8 = s_[3:6]
    s_ = apply(s_)
    u1s8 = s_[6:9]
    s_ = apply(s_)
    s_ = apply(s_)
    u2s8 = s_[9:12]
    for t in steps:
        t()

    def pool(v):
        return jnp.sum(v, axis=1, keepdims=True)

    cols = [pool(s8)]
    cols.append(pool(u0))
    cols.append(pool(jnp.abs(u0s1 - u0s2)))
    cols.append(pool(jnp.abs(u0s2 - u0s4)))
    cols.append(pool(jnp.abs(u0s4 - u0s8)))
    cols.append(pool(u1))
    cols.append(pool(jnp.abs(u1s2 - u1s4)))
    cols.append(pool(jnp.abs(u1s4 - u1s8)))
    cols.append(pool(u2))
    cols.append(pool(jnp.abs(u2s4 - u2s8)))
    cols.append(pool(jnp.abs(s4 - s8)))
    return jnp.concatenate(cols, axis=1)


def _block_kernel(xc_ref, xn_ref, out_ref, w_a, w_b, rd_a, rd_b):
    b = pl.program_id(0)
    xt_cur = xc_ref[0][0:_DIM]
    xt_nxt = xn_ref[0][0:_DIM]

    @pl.when(b == 0)
    def _():
        for t in _build_thunks(xt_cur, w_a, rd_a):
            t()

    @pl.when(b % 2 == 0)
    def _():
        out_ref[0] = _feats(xt_cur, w_a, rd_a,
                            _build_thunks(xt_nxt, w_b, rd_b))

    @pl.when(b % 2 == 1)
    def _():
        out_ref[0] = _feats(xt_cur, w_b, rd_b,
                            _build_thunks(xt_nxt, w_a, rd_a))


@functools.partial(jax.jit, static_argnames=())
def kernel(point_clouds, sigma, alphas):
    b_pc, n, dim = point_clouds.shape
    nw = alphas.shape[0]
    sqrt_sigma = jnp.sqrt(sigma.astype(jnp.float32))
    scale = alphas / sqrt_sigma
    xbt = (point_clouds.transpose(0, 2, 1)[:, None, :, :]
           * scale[None, :, :, None]).reshape(b_pc * nw, dim, n)
    xbt = jnp.pad(xbt, ((0, 0), (0, 8 - dim), (0, 0)))

    nb = b_pc * nw
    n_cols = 1 + (_J + 1) + (_J + 1) * _J // 2
    out = pl.pallas_call(
        _block_kernel,
        grid=(nb,),
        in_specs=[pl.BlockSpec((1, 8, n), lambda b: (b, 0, 0)),
                  pl.BlockSpec((1, 8, n), lambda b: (jnp.minimum(b + 1, _NB - 1), 0, 0))],
        out_specs=pl.BlockSpec((1, dim, n_cols), lambda b: (b, 0, 0)),
        out_shape=jax.ShapeDtypeStruct((nb, dim, n_cols), jnp.float32),
        scratch_shapes=[pltpu.VMEM((_N, _N), jnp.bfloat16),
                        pltpu.VMEM((_N, _N), jnp.bfloat16),
                        pltpu.VMEM((1, _N), jnp.float32),
                        pltpu.VMEM((1, _N), jnp.float32)],
    )(xbt, xbt)

    feats = out.transpose(0, 2, 1).reshape(nb, n_cols * dim)
    return (feats * sqrt_sigma).reshape(b_pc, nw * n_cols * dim)


# speedup vs baseline: 2.0346x; 2.0346x over previous
import functools

import jax
import jax.numpy as jnp
from jax.experimental import pallas as pl
from jax.experimental.pallas import tpu as pltpu

_DIM = 3
_THRESHOLD = 0.4
_J = 3
_N = 2048
_CHUNK = 256


def _block_kernel(xt_ref, out_ref, w_scr):
    xt = xt_ref[0]
    x_rows = [xt[d:d + 1, :] for d in range(_DIM)]
    deg_row = jnp.zeros((1, _N), jnp.float32)
    for c in range(_N // _CHUNK):
        sl = slice(c * _CHUNK, (c + 1) * _CHUNK)
        cols = [x_rows[d][:, sl].T for d in range(_DIM)]
        d0 = cols[0] - x_rows[0]
        D = d0 * d0
        for d in range(1, _DIM):
            dd = cols[d] - x_rows[d]
            D = D + dd * dd
        Wc = jnp.exp(-D)
        Wc = jnp.where(Wc >= _THRESHOLD, Wc, 0.0)
        deg_row = deg_row + jnp.sum(Wc, axis=0, keepdims=True)
        w_scr[sl, :] = Wc.astype(jnp.bfloat16)
    rdeg_row = 1.0 / jnp.maximum(deg_row, 1e-12)

    def apply(s):
        sp = (s * rdeg_row).astype(jnp.bfloat16)
        mv = jax.lax.dot_general(
            sp, w_scr[...], (((1,), (0,)), ((), ())),
            preferred_element_type=jnp.float32)
        return 0.5 * (s + mv)

    zeros9 = jnp.zeros((3 * _J, _N), jnp.float32)
    s_ = jnp.concatenate([xt, zeros9], axis=0)
    s_ = apply(s_)
    s1 = s_[0:3]
    u0 = jnp.abs(xt - s1)
    s_ = jnp.concatenate([s_[0:3], u0, s_[6:12]], axis=0)
    s_ = apply(s_)
    s2 = s_[0:3]
    u0s1 = s_[3:6]
    u1 = jnp.abs(s1 - s2)
    s_ = jnp.concatenate([s_[0:6], u1, s_[9:12]], axis=0)
    s_ = apply(s_)
    u0s2 = s_[3:6]
    s_ = apply(s_)
    s4, u1s2 = s_[0:3], s_[6:9]
    u2 = jnp.abs(s2 - s4)
    s_ = jnp.concatenate([s_[0:9], u2], axis=0)
    s_ = apply(s_)
    u0s4 = s_[3:6]
    s_ = apply(s_)
    u1s4 = s_[6:9]
    s_ = apply(s_)
    s_ = apply(s_)
    s8, u2s4 = s_[0:3], s_[9:12]
    s_ = apply(s_)
    u0s8 = s_[3:6]
    s_ = apply(s_)
    u1s8 = s_[6:9]
    s_ = apply(s_)
    s_ = apply(s_)
    u2s8 = s_[9:12]

    def pool(v):
        return jnp.sum(v, axis=1, keepdims=True)

    cols = [pool(s8)]
    cols.append(pool(u0))
    cols.append(pool(jnp.abs(u0s1 - u0s2)))
    cols.append(pool(jnp.abs(u0s2 - u0s4)))
    cols.append(pool(jnp.abs(u0s4 - u0s8)))
    cols.append(pool(u1))
    cols.append(pool(jnp.abs(u1s2 - u1s4)))
    cols.append(pool(jnp.abs(u1s4 - u1s8)))
    cols.append(pool(u2))
    cols.append(pool(jnp.abs(u2s4 - u2s8)))
    cols.append(pool(jnp.abs(s4 - s8)))
    out_ref[0] = jnp.concatenate(cols, axis=1)


@functools.partial(jax.jit, static_argnames=())
def kernel(point_clouds, sigma, alphas):
    b_pc, n, dim = point_clouds.shape
    nw = alphas.shape[0]
    sqrt_sigma = jnp.sqrt(sigma.astype(jnp.float32))
    scale = alphas / sqrt_sigma
    xbt = (point_clouds.transpose(0, 2, 1)[:, None, :, :]
           * scale[None, :, :, None]).reshape(b_pc * nw, dim, n)

    n_cols = 1 + (_J + 1) + (_J + 1) * _J // 2
    out = pl.pallas_call(
        _block_kernel,
        grid=(b_pc * nw,),
        in_specs=[pl.BlockSpec((1, dim, n), lambda b: (b, 0, 0))],
        out_specs=pl.BlockSpec((1, dim, n_cols), lambda b: (b, 0, 0)),
        out_shape=jax.ShapeDtypeStruct((b_pc * nw, dim, n_cols), jnp.float32),
        scratch_shapes=[pltpu.VMEM((_N, _N), jnp.bfloat16)],
    )(xbt)

    feats = out.transpose(0, 2, 1).reshape(b_pc * nw, n_cols * dim)
    return (feats * sqrt_sigma).reshape(b_pc, nw * n_cols * dim)


# speedup vs baseline: 2.3361x; 1.1482x over previous
import functools

import jax
import jax.numpy as jnp
from jax.experimental import pallas as pl
from jax.experimental.pallas import tpu as pltpu

_DIM = 3
_THRESHOLD = 0.4
_J = 3
_N = 2048
_CHUNK = 256


def _block_kernel(xt_ref, out_ref, w_scr):
    xt = xt_ref[0]
    rn_row = jnp.sum(xt * xt, axis=0, keepdims=True)

    deg_row = jnp.zeros((1, _N), jnp.float32)
    for c in range(_N // _CHUNK):
        xtc = xt[:, c * _CHUNK:(c + 1) * _CHUNK]
        rn_col = rn_row[:, c * _CHUNK:(c + 1) * _CHUNK].T
        G = jax.lax.dot_general(
            xtc, xt, (((0,), (0,)), ((), ())),
            preferred_element_type=jnp.float32)
        D = rn_col + rn_row - 2.0 * G
        Wc = jnp.exp(-D)
        Wc = jnp.where(Wc >= _THRESHOLD, Wc, 0.0)
        deg_row = deg_row + jnp.sum(Wc, axis=0, keepdims=True)
        w_scr[c * _CHUNK:(c + 1) * _CHUNK, :] = Wc.astype(jnp.bfloat16)
    rdeg_row = 1.0 / jnp.maximum(deg_row, 1e-12)

    def apply(s):
        sp = (s * rdeg_row).astype(jnp.bfloat16)
        mv = jax.lax.dot_general(
            sp, w_scr[...], (((1,), (0,)), ((), ())),
            preferred_element_type=jnp.float32)
        return 0.5 * (s + mv)

    zeros13 = jnp.zeros((13, _N), jnp.float32)
    s_ = jnp.concatenate([xt, zeros13], axis=0)
    s_ = apply(s_)
    s1 = s_[0:3]
    u0 = jnp.abs(xt - s1)
    s_ = jnp.concatenate([s_[0:3], u0, s_[6:16]], axis=0)
    s_ = apply(s_)
    s2 = s_[0:3]
    u0s1 = s_[3:6]
    u1 = jnp.abs(s1 - s2)
    s_ = jnp.concatenate([s_[0:6], u1, s_[9:16]], axis=0)
    s_ = apply(s_)
    u0s2 = s_[3:6]
    s_ = apply(s_)
    s4, u1s2 = s_[0:3], s_[6:9]
    u2 = jnp.abs(s2 - s4)
    s_ = jnp.concatenate([s_[0:9], u2, s_[12:16]], axis=0)
    s_ = apply(s_)
    u0s4 = s_[3:6]
    s_ = apply(s_)
    u1s4 = s_[6:9]
    s_ = apply(s_)
    s_ = apply(s_)
    s8, u2s4 = s_[0:3], s_[9:12]
    s_ = apply(s_)
    u0s8 = s_[3:6]
    s_ = apply(s_)
    u1s8 = s_[6:9]
    s_ = apply(s_)
    s_ = apply(s_)
    u2s8 = s_[9:12]

    def pool(v):
        return jnp.sum(v, axis=1, keepdims=True)

    cols = [pool(s8)]
    cols.append(pool(u0))
    cols.append(pool(jnp.abs(u0s1 - u0s2)))
    cols.append(pool(jnp.abs(u0s2 - u0s4)))
    cols.append(pool(jnp.abs(u0s4 - u0s8)))
    cols.append(pool(u1))
    cols.append(pool(jnp.abs(u1s2 - u1s4)))
    cols.append(pool(jnp.abs(u1s4 - u1s8)))
    cols.append(pool(u2))
    cols.append(pool(jnp.abs(u2s4 - u2s8)))
    cols.append(pool(jnp.abs(s4 - s8)))
    out_ref[0] = jnp.concatenate(cols, axis=1)


@functools.partial(jax.jit, static_argnames=())
def kernel(point_clouds, sigma, alphas):
    b_pc, n, dim = point_clouds.shape
    nw = alphas.shape[0]
    sqrt_sigma = jnp.sqrt(sigma.astype(jnp.float32))
    scale = alphas / sqrt_sigma
    xbt = (point_clouds.transpose(0, 2, 1)[:, None, :, :]
           * scale[None, :, :, None]).reshape(b_pc * nw, dim, n)

    n_cols = 1 + (_J + 1) + (_J + 1) * _J // 2
    out = pl.pallas_call(
        _block_kernel,
        grid=(b_pc * nw,),
        in_specs=[pl.BlockSpec((1, dim, n), lambda b: (b, 0, 0))],
        out_specs=pl.BlockSpec((1, dim, n_cols), lambda b: (b, 0, 0)),
        out_shape=jax.ShapeDtypeStruct((b_pc * nw, dim, n_cols), jnp.float32),
        scratch_shapes=[pltpu.VMEM((_N, _N), jnp.bfloat16)],
    )(xbt)

    feats = out.transpose(0, 2, 1).reshape(b_pc * nw, n_cols * dim)
    return (feats * sqrt_sigma).reshape(b_pc, nw * n_cols * dim)
